# baseline (speedup 1.0000x reference)
import functools

import jax
import jax.numpy as jnp
from jax import lax
from jax.experimental import pallas as pl
from jax.experimental.pallas import tpu as pltpu
from jax.experimental.pallas import tpu_sc as plsc

_NW = 32
_VOCAB = 1000001
_VP = 1000064
_NCOLS = _VP // 128
_COLS_PW = 246
_PACK_ROWS = _VP // 2
_D = 64
_MAX_POS = 1024

_CPARAMS = pltpu.CompilerParams(use_tc_tiling_on_sc=True, needs_layout_passes=False)


def _pos_freqs(nt: int) -> jnp.ndarray:
    dim = _D
    inv = 1.0 / (10000.0 ** (jnp.arange(0, dim, 2)[: dim // 2].astype(jnp.float32) / dim))
    pos = jnp.minimum(jnp.arange(nt, dtype=jnp.float32), float(_MAX_POS - 1))
    f = pos[:, None] * inv[None, :]
    return jnp.concatenate([jnp.cos(f), jnp.sin(f)], axis=-1)


def _pack_body(wt_hbm, packed_hbm, bi0, bi1, bo0, bo1, si0, si1, so0, so1):
    wid = lax.axis_index("s") * 2 + lax.axis_index("c")
    c0 = wid * _COLS_PW
    iota = lax.iota(jnp.int32, 16)
    rows = [iota + (16 * g % 64) for g in range(8)]
    bins = (bi0, bi1)
    bouts = (bo0, bo1)
    sins = (si0, si1)
    souts = (so0, so1)

    def col_of(i):
        return lax.rem(c0 + i, _NCOLS)

    def start_in(i, b):
        pltpu.async_copy(wt_hbm.at[:, pl.ds(col_of(i) * 128, 128)],
                         bins[b].at[:, pl.ds(0, 128)], sins[b])

    def wait_in(b):
        pltpu.make_async_copy(wt_hbm.at[:, pl.ds(0, 128)],
                              bins[b].at[:, pl.ds(0, 128)], sins[b]).wait()

    def wait_out(b):
        pltpu.make_async_copy(packed_hbm.at[pl.ds(0, 64)], bouts[b], souts[b]).wait()

    def do_col(i, b, first):
        wait_in(b)
        if not first:
            wait_out(b)

        def u_body(u, c_even):
            c_odd = c_even + 1
            for g in range(8):
                col = c_even if g < 4 else c_odd
                v = plsc.load_gather(bins[b], [rows[g], col])
                bouts[b][u, pl.ds(g * 16, 16)] = v
            return c_even + 2

        lax.fori_loop(0, 64, u_body, jnp.zeros((16,), jnp.int32))
        pltpu.async_copy(bouts[b], packed_hbm.at[pl.ds(col_of(i) * 64, 64)], souts[b])

    start_in(0, 0)
    start_in(1, 1)
    do_col(0, 0, first=True)
    start_in(2, 0)
    do_col(1, 1, first=True)

    def pair_body(p, carry):
        i = 2 * p

        @pl.when(i + 1 < _COLS_PW)
        def _():
            start_in(i + 1, 1)

        do_col(i, 0, first=False)

        @pl.when(i + 1 < _COLS_PW)
        def _():
            @pl.when(i + 2 < _COLS_PW)
            def _():
                start_in(i + 2, 0)

            do_col(i + 1, 1, first=False)

        return carry

    lax.fori_loop(1, _COLS_PW // 2, pair_body, 0)
    wait_out(0)
    wait_out(1)


def _gather_body(p_hbm, v_hbm, fq_hbm, out_hbm, v_v, u_v, fqb_v,
                 g0, g1, o0, o1, sg0, sg1, so0, so1):
    wid = lax.axis_index("s") * 2 + lax.axis_index("c")
    pltpu.sync_copy(v_hbm.at[wid], v_v)
    iota = lax.iota(jnp.int32, 16)

    def shift_body(r, carry):
        for g in range(8):
            sl = pl.ds(g * 16, 16)
            u_v[r, sl] = lax.shift_right_logical(v_v[r, sl], 1)
        return carry

    lax.fori_loop(0, 50, shift_body, 0)

    gbuf = (g0, g1)
    obuf = (o0, o1)
    gsem = (sg0, sg1)
    osem = (so0, so1)

    def start_gather(k, b):
        pltpu.async_copy(p_hbm.at[u_v.at[k]], gbuf[b].at[:, pl.ds(0, 128)],
                         gsem[b])

    def wait_gather(b):
        pltpu.make_async_copy(p_hbm.at[u_v.at[0]],
                              gbuf[b].at[:, pl.ds(0, 128)], gsem[b]).wait()

    def wait_out(b):
        pltpu.make_async_copy(p_hbm.at[pl.ds(0, 64)], obuf[b], osem[b]).wait()

    start_gather(0, 0)

    def blk(k, b, first):
        blk_id = wid * 50 + k
        t = blk_id // 8
        bb = lax.rem(blk_id, 8)

        @pl.when(k + 1 < 50)
        def _():
            start_gather(k + 1, 1 - b)

        wait_gather(b)
        if not first:
            wait_out(b)
        pltpu.sync_copy(fq_hbm.at[t], fqb_v)
        rows_g = [iota + 16 * g for g in range(8)]
        par64 = []
        for g in range(8):
            sl = pl.ds(g * 16, 16)
            par64.append(jnp.bitwise_and(v_v[k, sl], 1) * 64)

        def d_body(d, dvec):
            fqs = fqb_v[d // 8, pl.ds(lax.rem(d, 8) * 16, 16)]
            for g in range(8):
                col = par64[g] + dvec
                val = plsc.load_gather(gbuf[b], [rows_g[g], col])
                obuf[b][d, pl.ds(g * 16, 16)] = val + fqs
            return dvec + 1

        lax.fori_loop(0, 64, d_body, jnp.zeros((16,), jnp.int32))
        pltpu.async_copy(obuf[b], out_hbm.at[t, :, pl.ds(bb * 128, 128)], osem[b])

    blk(0, 0, first=True)
    blk(1, 1, first=True)

    def outer(i, carry):
        blk(2 * i, 0, first=False)
        blk(2 * i + 1, 1, first=False)
        return carry

    lax.fori_loop(1, 25, outer, 0)
    wait_out(0)
    wait_out(1)


def kernel(text, text_embed_weight):
    b, nt = text.shape
    vocab, d = text_embed_weight.shape
    assert (b, nt, vocab, d) == (1024, 200, _VOCAB, _D)

    mesh = plsc.VectorSubcoreMesh(core_axis_name="c", subcore_axis_name="s")

    wt = text_embed_weight.T
    v3 = (text.astype(jnp.int32) + 1).T.reshape(_NW, 50, 128)
    fqb = jnp.repeat(_pos_freqs(nt), 16, axis=1).reshape(nt, 8, 128)

    pack = functools.partial(
        pl.kernel,
        mesh=mesh,
        compiler_params=_CPARAMS,
        out_type=jax.ShapeDtypeStruct((_PACK_ROWS, 128), jnp.float32),
        scratch_types=[
            pltpu.VMEM((64, 129), jnp.float32),
            pltpu.VMEM((64, 129), jnp.float32),
            pltpu.VMEM((64, 128), jnp.float32),
            pltpu.VMEM((64, 128), jnp.float32),
            pltpu.SemaphoreType.DMA,
            pltpu.SemaphoreType.DMA,
            pltpu.SemaphoreType.DMA,
            pltpu.SemaphoreType.DMA,
        ],
    )(_pack_body)
    packed = pack(wt)

    gather = functools.partial(
        pl.kernel,
        mesh=mesh,
        compiler_params=_CPARAMS,
        out_type=jax.ShapeDtypeStruct((nt, d, b), jnp.float32),
        scratch_types=[
            pltpu.VMEM((50, 128), jnp.int32),
            pltpu.VMEM((50, 128), jnp.int32),
            pltpu.VMEM((8, 128), jnp.float32),
            pltpu.VMEM((128, 129), jnp.float32),
            pltpu.VMEM((128, 129), jnp.float32),
            pltpu.VMEM((64, 128), jnp.float32),
            pltpu.VMEM((64, 128), jnp.float32),
            pltpu.SemaphoreType.DMA,
            pltpu.SemaphoreType.DMA,
            pltpu.SemaphoreType.DMA,
            pltpu.SemaphoreType.DMA,
        ],
    )(_gather_body)
    out = gather(packed, v3, fqb)
    return jnp.transpose(out, (2, 0, 1))


# speedup vs baseline: 1.0225x; 1.0225x over previous
import functools

import jax
import jax.numpy as jnp
from jax import lax
from jax.experimental import pallas as pl
from jax.experimental.pallas import tpu as pltpu
from jax.experimental.pallas import tpu_sc as plsc

_NW = 32
_VOCAB = 1000001
_VP = 1000064
_NCOLS = _VP // 128
_COLS_PW = 246
_PACK_ROWS = _VP // 2
_D = 64
_MAX_POS = 1024

_CPARAMS = pltpu.CompilerParams(use_tc_tiling_on_sc=True, needs_layout_passes=False)


def _pos_freqs(nt: int) -> jnp.ndarray:
    dim = _D
    inv = 1.0 / (10000.0 ** (jnp.arange(0, dim, 2)[: dim // 2].astype(jnp.float32) / dim))
    pos = jnp.minimum(jnp.arange(nt, dtype=jnp.float32), float(_MAX_POS - 1))
    f = pos[:, None] * inv[None, :]
    return jnp.concatenate([jnp.cos(f), jnp.sin(f)], axis=-1)


def _pack_body(wt_hbm, packed_hbm, bi0, bi1, bi2, bo0, bo1, bo2,
               si0, si1, si2, so0, so1, so2):
    wid = lax.axis_index("s") * 2 + lax.axis_index("c")
    c0 = wid * _COLS_PW
    iota = lax.iota(jnp.int32, 16)
    rows = [iota + (16 * g % 64) for g in range(8)]
    bins = (bi0, bi1, bi2)
    bouts = (bo0, bo1, bo2)
    sins = (si0, si1, si2)
    souts = (so0, so1, so2)

    def col_of(i):
        return lax.rem(c0 + i, _NCOLS)

    def start_in(i, b):
        pltpu.async_copy(wt_hbm.at[:, pl.ds(col_of(i) * 128, 128)],
                         bins[b], sins[b])

    def wait_in(b):
        pltpu.make_async_copy(wt_hbm.at[:, pl.ds(0, 128)],
                              bins[b], sins[b]).wait()

    def wait_out(b):
        pltpu.make_async_copy(packed_hbm.at[pl.ds(0, 64)], bouts[b], souts[b]).wait()

    def do_col(i, b, first):
        wait_in(b)
        if not first:
            wait_out(b)

        def u_body(u, c_even):
            c_odd = c_even + 1
            for g in range(8):
                col = c_even if g < 4 else c_odd
                v = plsc.load_gather(bins[b], [rows[g], col])
                bouts[b][u, pl.ds(g * 16, 16)] = v
            return c_even + 2

        lax.fori_loop(0, 64, u_body, jnp.zeros((16,), jnp.int32))

        pltpu.async_copy(bouts[b], packed_hbm.at[pl.ds(col_of(i) * 64, 64)], souts[b])

    start_in(0, 0)
    start_in(1, 1)
    start_in(2, 2)
    do_col(0, 0, first=True)
    start_in(3, 0)
    do_col(1, 1, first=True)
    start_in(4, 1)
    do_col(2, 2, first=True)

    def tri_body2(p, carry):
        i = 3 * p
        for j in range(3):
            @pl.when(i + j + 2 < _COLS_PW)
            def _():
                start_in(i + j + 2, (j + 2) % 3)

            do_col(i + j, j, first=False)
        return carry

    lax.fori_loop(1, _COLS_PW // 3, tri_body2, 0)
    wait_out(0)
    wait_out(1)
    wait_out(2)


def _gather_body(p_hbm, v_hbm, fq_hbm, out_hbm, v_v, u_v, fq_v,
                 g0, g1, g2, o0, o1, o2,
                 sg0, sg1, sg2, so0, so1, so2):
    wid = lax.axis_index("s") * 2 + lax.axis_index("c")
    t0 = (wid * 50) // 8
    pltpu.sync_copy(v_hbm.at[wid], v_v)
    pltpu.sync_copy(fq_hbm.at[pl.ds(t0, 8)], fq_v)
    iota = lax.iota(jnp.int32, 16)
    rows_g = [iota + 16 * g for g in range(8)]

    def shift_body(r, carry):
        for g in range(8):
            sl = pl.ds(g * 16, 16)
            u_v[r, sl] = lax.shift_right_logical(v_v[r, sl], 1)
        return carry

    lax.fori_loop(0, 50, shift_body, 0)

    gbuf = (g0, g1, g2)
    obuf = (o0, o1, o2)
    gsem = (sg0, sg1, sg2)
    osem = (so0, so1, so2)

    def start_gather(k, b):
        pltpu.async_copy(p_hbm.at[u_v.at[k]], gbuf[b], gsem[b])

    def wait_gather(b):
        pltpu.make_async_copy(p_hbm.at[u_v.at[0]], gbuf[b], gsem[b]).wait()

    def wait_out(b):
        pltpu.make_async_copy(p_hbm.at[pl.ds(0, 64)], obuf[b], osem[b]).wait()

    start_gather(0, 0)
    start_gather(1, 1)

    def blk(k, b, first):
        blk_id = wid * 50 + k
        t = blk_id // 8
        bb = lax.rem(blk_id, 8)

        @pl.when(k + 2 < 50)
        def _():
            start_gather(k + 2, (b + 2) % 3)

        wait_gather(b)
        if not first:
            wait_out(b)
        tl = t - t0
        par64 = []
        for g in range(8):
            sl = pl.ds(g * 16, 16)
            par64.append(jnp.bitwise_and(v_v[k, sl], 1) * 64)

        def d_body(d, dvec):
            fqs = fq_v[tl, d // 8, pl.ds(lax.rem(d, 8) * 16, 16)]
            for g in range(8):
                col = par64[g] + dvec
                val = plsc.load_gather(gbuf[b], [rows_g[g], col])
                obuf[b][d, pl.ds(g * 16, 16)] = val + fqs
            return dvec + 1

        lax.fori_loop(0, 64, d_body, jnp.zeros((16,), jnp.int32))

        pltpu.async_copy(obuf[b], out_hbm.at[t, :, pl.ds(bb * 128, 128)], osem[b])

    blk(0, 0, first=True)
    blk(1, 1, first=True)
    blk(2, 2, first=True)

    def tri(i, carry):
        for j in range(3):
            blk(3 * i + j, j, first=False)
        return carry

    lax.fori_loop(1, 16, tri, 0)
    blk(48, 0, first=False)
    blk(49, 1, first=False)
    wait_out(0)
    wait_out(1)
    wait_out(2)


def kernel(text, text_embed_weight):
    b, nt = text.shape
    vocab, d = text_embed_weight.shape
    assert (b, nt, vocab, d) == (1024, 200, _VOCAB, _D)

    mesh = plsc.VectorSubcoreMesh(core_axis_name="c", subcore_axis_name="s")

    wt = text_embed_weight.T
    v3 = (text.astype(jnp.int32) + 1).T.reshape(_NW, 50, 128)
    fqb = jnp.pad(jnp.repeat(_pos_freqs(nt), 16, axis=1).reshape(nt, 8, 128),
                  ((0, 8), (0, 0), (0, 0)))

    pack = functools.partial(
        pl.kernel,
        mesh=mesh,
        compiler_params=_CPARAMS,
        out_type=jax.ShapeDtypeStruct((_PACK_ROWS, 128), jnp.float32),
        scratch_types=[
            pltpu.VMEM((64, 128), jnp.float32),
            pltpu.VMEM((64, 128), jnp.float32),
            pltpu.VMEM((64, 128), jnp.float32),
            pltpu.VMEM((64, 128), jnp.float32),
            pltpu.VMEM((64, 128), jnp.float32),
            pltpu.VMEM((64, 128), jnp.float32),
            pltpu.SemaphoreType.DMA,
            pltpu.SemaphoreType.DMA,
            pltpu.SemaphoreType.DMA,
            pltpu.SemaphoreType.DMA,
            pltpu.SemaphoreType.DMA,
            pltpu.SemaphoreType.DMA,
        ],
    )(_pack_body)
    packed = pack(wt)

    gather = functools.partial(
        pl.kernel,
        mesh=mesh,
        compiler_params=_CPARAMS,
        out_type=jax.ShapeDtypeStruct((nt, d, b), jnp.float32),
        scratch_types=[
            pltpu.VMEM((50, 128), jnp.int32),
            pltpu.VMEM((50, 128), jnp.int32),
            pltpu.VMEM((8, 8, 128), jnp.float32),
            pltpu.VMEM((128, 128), jnp.float32),
            pltpu.VMEM((128, 128), jnp.float32),
            pltpu.VMEM((128, 128), jnp.float32),
            pltpu.VMEM((64, 128), jnp.float32),
            pltpu.VMEM((64, 128), jnp.float32),
            pltpu.VMEM((64, 128), jnp.float32),
            pltpu.SemaphoreType.DMA,
            pltpu.SemaphoreType.DMA,
            pltpu.SemaphoreType.DMA,
            pltpu.SemaphoreType.DMA,
            pltpu.SemaphoreType.DMA,
            pltpu.SemaphoreType.DMA,
        ],
    )(_gather_body)
    out = gather(packed, v3, fqb)
    return jnp.transpose(out, (2, 0, 1))


# speedup vs baseline: 1.3545x; 1.3247x over previous
import functools

import jax
import jax.numpy as jnp
from jax import lax
from jax.experimental import pallas as pl
from jax.experimental.pallas import tpu as pltpu
from jax.experimental.pallas import tpu_sc as plsc

_NW = 32
_VOCAB = 1000001
_VP = 1000064
_NCOLS = _VP // 128
_COLS_PW = 246
_PACK_ROWS = _VP // 2
_D = 64
_MAX_POS = 1024

_CPARAMS = pltpu.CompilerParams(use_tc_tiling_on_sc=True, needs_layout_passes=False)


def _pos_freqs(nt: int) -> jnp.ndarray:
    dim = _D
    inv = 1.0 / (10000.0 ** (jnp.arange(0, dim, 2)[: dim // 2].astype(jnp.float32) / dim))
    pos = jnp.minimum(jnp.arange(nt, dtype=jnp.float32), float(_MAX_POS - 1))
    f = pos[:, None] * inv[None, :]
    return jnp.concatenate([jnp.cos(f), jnp.sin(f)], axis=-1)


def _pack_body(wt_hbm, packed_hbm, bi0, bi1, bi2, bo0, bo1, bo2,
               si0, si1, si2, so0, so1, so2):
    wid = lax.axis_index("s") * 2 + lax.axis_index("c")
    c0 = wid * _COLS_PW
    iota = lax.iota(jnp.int32, 16)
    rows = [iota + (16 * g % 64) for g in range(8)]
    bins = (bi0, bi1, bi2)
    bouts = (bo0, bo1, bo2)
    sins = (si0, si1, si2)
    souts = (so0, so1, so2)

    def col_of(i):
        return lax.rem(c0 + i, _NCOLS)

    def start_in(i, b):
        pltpu.async_copy(wt_hbm.at[:, pl.ds(col_of(i) * 128, 128)],
                         bins[b], sins[b])

    def wait_in(b):
        pltpu.make_async_copy(wt_hbm.at[:, pl.ds(0, 128)],
                              bins[b], sins[b]).wait()

    def wait_out(b):
        pltpu.make_async_copy(packed_hbm.at[pl.ds(0, 64)], bouts[b], souts[b]).wait()

    def do_col(i, b, first):
        wait_in(b)
        if not first:
            wait_out(b)

        def load_u(c_even):
            c_odd = c_even + 1
            return tuple(
                plsc.load_gather(bins[b], [rows[g], c_even if g < 4 else c_odd])
                for g in range(8))

        def u_body(u, carry):
            c_even, vecs = carry
            new = load_u(c_even)
            for g in range(8):
                bouts[b][u - 1, pl.ds(g * 16, 16)] = vecs[g]
            return (c_even + 2, new)

        first_vecs = load_u(jnp.zeros((16,), jnp.int32))
        c_last, last_vecs = lax.fori_loop(
            1, 64, u_body, (jnp.full((16,), 2, jnp.int32), first_vecs))
        for g in range(8):
            bouts[b][63, pl.ds(g * 16, 16)] = last_vecs[g]

        pltpu.async_copy(bouts[b], packed_hbm.at[pl.ds(col_of(i) * 64, 64)], souts[b])

    start_in(0, 0)
    start_in(1, 1)
    start_in(2, 2)
    do_col(0, 0, first=True)
    start_in(3, 0)
    do_col(1, 1, first=True)
    start_in(4, 1)
    do_col(2, 2, first=True)

    def tri_body2(p, carry):
        i = 3 * p
        for j in range(3):
            @pl.when(i + j + 2 < _COLS_PW)
            def _():
                start_in(i + j + 2, (j + 2) % 3)

            do_col(i + j, j, first=False)
        return carry

    lax.fori_loop(1, _COLS_PW // 3, tri_body2, 0)
    wait_out(0)
    wait_out(1)
    wait_out(2)


def _gather_body(p_hbm, v_hbm, fq_hbm, out_hbm, v_v, u_v, fq_v,
                 g0, g1, g2, o0, o1, o2,
                 sg0, sg1, sg2, so0, so1, so2):
    wid = lax.axis_index("s") * 2 + lax.axis_index("c")
    t0 = (wid * 50) // 8
    pltpu.sync_copy(v_hbm.at[wid], v_v)
    pltpu.sync_copy(fq_hbm.at[pl.ds(t0, 8)], fq_v)
    iota = lax.iota(jnp.int32, 16)
    rows_g = [iota + 16 * g for g in range(8)]

    def shift_body(r, carry):
        for g in range(8):
            sl = pl.ds(g * 16, 16)
            u_v[r, sl] = lax.shift_right_logical(v_v[r, sl], 1)
        return carry

    lax.fori_loop(0, 50, shift_body, 0)

    gbuf = (g0, g1, g2)
    obuf = (o0, o1, o2)
    gsem = (sg0, sg1, sg2)
    osem = (so0, so1, so2)

    def start_gather(k, b):
        pltpu.async_copy(p_hbm.at[u_v.at[k]], gbuf[b], gsem[b])

    def wait_gather(b):
        pltpu.make_async_copy(p_hbm.at[u_v.at[0]], gbuf[b], gsem[b]).wait()

    def wait_out(b):
        pltpu.make_async_copy(p_hbm.at[pl.ds(0, 64)], obuf[b], osem[b]).wait()

    start_gather(0, 0)
    start_gather(1, 1)

    def blk(k, b, first):
        blk_id = wid * 50 + k
        t = blk_id // 8
        bb = lax.rem(blk_id, 8)

        @pl.when(k + 2 < 50)
        def _():
            start_gather(k + 2, (b + 2) % 3)

        wait_gather(b)
        if not first:
            wait_out(b)
        tl = t - t0
        par64 = []
        for g in range(8):
            sl = pl.ds(g * 16, 16)
            par64.append(jnp.bitwise_and(v_v[k, sl], 1) * 64)

        def load_d(d, dvec):
            fqs = fq_v[tl, d // 8, pl.ds(lax.rem(d, 8) * 16, 16)]
            return fqs, tuple(
                plsc.load_gather(gbuf[b], [rows_g[g], par64[g] + dvec])
                for g in range(8))

        def d_body(d, carry):
            dvec, fqs, vecs = carry
            nfqs, new = load_d(d, dvec)
            for g in range(8):
                obuf[b][d - 1, pl.ds(g * 16, 16)] = vecs[g] + fqs
            return (dvec + 1, nfqs, new)

        fqs0, vecs0 = load_d(0, jnp.zeros((16,), jnp.int32))
        _, fqs_l, vecs_l = lax.fori_loop(
            1, 64, d_body, (jnp.full((16,), 1, jnp.int32), fqs0, vecs0))
        for g in range(8):
            obuf[b][63, pl.ds(g * 16, 16)] = vecs_l[g] + fqs_l

        pltpu.async_copy(obuf[b], out_hbm.at[t, :, pl.ds(bb * 128, 128)], osem[b])

    blk(0, 0, first=True)
    blk(1, 1, first=True)
    blk(2, 2, first=True)

    def tri(i, carry):
        for j in range(3):
            blk(3 * i + j, j, first=False)
        return carry

    lax.fori_loop(1, 16, tri, 0)
    blk(48, 0, first=False)
    blk(49, 1, first=False)
    wait_out(0)
    wait_out(1)
    wait_out(2)


def kernel(text, text_embed_weight):
    b, nt = text.shape
    vocab, d = text_embed_weight.shape
    assert (b, nt, vocab, d) == (1024, 200, _VOCAB, _D)

    mesh = plsc.VectorSubcoreMesh(core_axis_name="c", subcore_axis_name="s")

    wt = text_embed_weight.T
    v3 = (text.astype(jnp.int32) + 1).T.reshape(_NW, 50, 128)
    fqb = jnp.pad(jnp.repeat(_pos_freqs(nt), 16, axis=1).reshape(nt, 8, 128),
                  ((0, 8), (0, 0), (0, 0)))

    pack = functools.partial(
        pl.kernel,
        mesh=mesh,
        compiler_params=_CPARAMS,
        out_type=jax.ShapeDtypeStruct((_PACK_ROWS, 128), jnp.float32),
        scratch_types=[
            pltpu.VMEM((64, 128), jnp.float32),
            pltpu.VMEM((64, 128), jnp.float32),
            pltpu.VMEM((64, 128), jnp.float32),
            pltpu.VMEM((64, 128), jnp.float32),
            pltpu.VMEM((64, 128), jnp.float32),
            pltpu.VMEM((64, 128), jnp.float32),
            pltpu.SemaphoreType.DMA,
            pltpu.SemaphoreType.DMA,
            pltpu.SemaphoreType.DMA,
            pltpu.SemaphoreType.DMA,
            pltpu.SemaphoreType.DMA,
            pltpu.SemaphoreType.DMA,
        ],
    )(_pack_body)
    packed = pack(wt)

    gather = functools.partial(
        pl.kernel,
        mesh=mesh,
        compiler_params=_CPARAMS,
        out_type=jax.ShapeDtypeStruct((nt, d, b), jnp.float32),
        scratch_types=[
            pltpu.VMEM((50, 128), jnp.int32),
            pltpu.VMEM((50, 128), jnp.int32),
            pltpu.VMEM((8, 8, 128), jnp.float32),
            pltpu.VMEM((128, 128), jnp.float32),
            pltpu.VMEM((128, 128), jnp.float32),
            pltpu.VMEM((128, 128), jnp.float32),
            pltpu.VMEM((64, 128), jnp.float32),
            pltpu.VMEM((64, 128), jnp.float32),
            pltpu.VMEM((64, 128), jnp.float32),
            pltpu.SemaphoreType.DMA,
            pltpu.SemaphoreType.DMA,
            pltpu.SemaphoreType.DMA,
            pltpu.SemaphoreType.DMA,
            pltpu.SemaphoreType.DMA,
            pltpu.SemaphoreType.DMA,
        ],
    )(_gather_body)
    out = gather(packed, v3, fqb)
    return jnp.transpose(out, (2, 0, 1))
